# baseline (speedup 1.0000x reference)
import jax
import jax.numpy as jnp
from jax import lax
from jax.experimental import pallas as pl
from jax.experimental.pallas import tpu as pltpu
from jax.experimental.pallas import tpu_sc as plsc

N = 100000
D = 128
NW = 32
C = 320
NCH = 10
RPW = NCH * C
NPAD = NW * RPW
NPAIR = (NCH - 2) // 2
NPAIR_LAST = 1
LAST_FULL = 2
TAIL = N - (NW - 1) * RPW - LAST_FULL * C


def _gather_body(x_hbm, idx_hbm, out_hbm, idx_v, buf0, buf1, g0, g1):
    wid = lax.axis_index("s") * 2 + lax.axis_index("c")
    base = pl.multiple_of(wid * RPW, RPW)
    last = wid == NW - 1
    pltpu.sync_copy(idx_hbm.at[pl.ds(base, RPW)], idx_v)

    def gather(k, buf, sem):
        off = pl.multiple_of(k * C, C)
        return pltpu.async_copy(x_hbm.at[idx_v.at[pl.ds(off, C)]], buf, sem)

    def gwait(k, buf, sem):
        off = pl.multiple_of(k * C, C)
        pltpu.make_async_copy(x_hbm.at[idx_v.at[pl.ds(off, C)]], buf, sem).wait()

    def store(k, buf):
        pltpu.sync_copy(buf, out_hbm.at[pl.ds(base + k * C, C)])

    npair = jnp.where(last, NPAIR_LAST, NPAIR)
    gather(0, buf0, g0)

    def pair(i, carry):
        k0 = 2 * i
        gather(k0 + 1, buf1, g1)
        gwait(k0, buf0, g0)
        store(k0, buf0)
        gather(k0 + 2, buf0, g0)
        gwait(k0 + 1, buf1, g1)
        store(k0 + 1, buf1)
        return carry

    lax.fori_loop(0, npair, pair, 0)

    @pl.when(jnp.logical_not(last))
    def _():
        gather(2 * NPAIR + 1, buf1, g1)
        gwait(2 * NPAIR, buf0, g0)
        store(2 * NPAIR, buf0)
        gwait(2 * NPAIR + 1, buf1, g1)
        store(2 * NPAIR + 1, buf1)

    @pl.when(last)
    def _():
        gwait(LAST_FULL, buf0, g0)
        pltpu.sync_copy(
            buf0.at[pl.ds(0, TAIL)],
            out_hbm.at[pl.ds(base + LAST_FULL * C, TAIL)],
        )


@jax.jit
def _gather(x, idx):
    mesh = plsc.VectorSubcoreMesh(core_axis_name="c", subcore_axis_name="s")
    f = pl.kernel(
        _gather_body,
        out_type=jax.ShapeDtypeStruct((N, D), jnp.float32),
        mesh=mesh,
        scratch_types=[
            pltpu.VMEM((RPW,), jnp.int32),
            pltpu.VMEM((C, D), jnp.float32),
            pltpu.VMEM((C, D), jnp.float32),
            pltpu.SemaphoreType.DMA,
            pltpu.SemaphoreType.DMA,
        ],
    )
    return f(x, idx)


def kernel(x, cell_type_indices, permutations):
    idx = permutations.reshape(-1).astype(jnp.int32)
    idx = jnp.concatenate([idx, jnp.zeros((NPAD - N,), jnp.int32)])
    return _gather(x, idx)


# speedup vs baseline: 1.3087x; 1.3087x over previous
import jax
import jax.numpy as jnp
from jax import lax
from jax.experimental import pallas as pl
from jax.experimental.pallas import tpu as pltpu
from jax.experimental.pallas import tpu_sc as plsc

N = 100000
D = 128
NW = 32
C = 256
NCH = 13
RPW = NCH * C
NPAD = NW * RPW
NPAIR = (NCH - 1) // 2
W30 = 30
P30A = 128
P30B = 160 - P30A


def _gather_body(x_hbm, idx_hbm, out_hbm, idx_v, buf0, buf1, g0, g1):
    wid = lax.axis_index("s") * 2 + lax.axis_index("c")
    base = pl.multiple_of(wid * RPW, RPW)

    def gather(k, buf, sem):
        off = pl.multiple_of(k * C, C)
        return pltpu.async_copy(x_hbm.at[idx_v.at[pl.ds(off, C)]], buf, sem)

    def gwait(k, buf, sem):
        off = pl.multiple_of(k * C, C)
        pltpu.make_async_copy(x_hbm.at[idx_v.at[pl.ds(off, C)]], buf, sem).wait()

    def store(k, buf):
        pltpu.sync_copy(buf, out_hbm.at[pl.ds(base + k * C, C)])

    @pl.when(wid < W30)
    def _():
        pltpu.sync_copy(idx_hbm.at[pl.ds(base, RPW)], idx_v)
        gather(0, buf0, g0)

        def pair(i, carry):
            k0 = 2 * i
            gather(k0 + 1, buf1, g1)
            gwait(k0, buf0, g0)
            store(k0, buf0)
            gather(k0 + 2, buf0, g0)
            gwait(k0 + 1, buf1, g1)
            store(k0 + 1, buf1)
            return carry

        lax.fori_loop(0, NPAIR, pair, 0)
        gwait(2 * NPAIR, buf0, g0)
        store(2 * NPAIR, buf0)

    @pl.when(wid == W30)
    def _():
        pltpu.sync_copy(idx_hbm.at[pl.ds(base, C)], idx_v.at[pl.ds(0, C)])
        pltpu.async_copy(
            x_hbm.at[idx_v.at[pl.ds(0, P30A)]],
            buf0.at[pl.ds(0, P30A)], g0).wait()
        pltpu.sync_copy(
            buf0.at[pl.ds(0, P30A)], out_hbm.at[pl.ds(base, P30A)])
        pltpu.async_copy(
            x_hbm.at[idx_v.at[pl.ds(P30A, P30B)]],
            buf0.at[pl.ds(0, P30B)], g0).wait()
        pltpu.sync_copy(
            buf0.at[pl.ds(0, P30B)],
            out_hbm.at[pl.ds(base + P30A, P30B)])


@jax.jit
def _gather(x, idx):
    mesh = plsc.VectorSubcoreMesh(core_axis_name="c", subcore_axis_name="s")
    f = pl.kernel(
        _gather_body,
        out_type=jax.ShapeDtypeStruct((N, D), jnp.float32),
        mesh=mesh,
        scratch_types=[
            pltpu.VMEM((RPW,), jnp.int32),
            pltpu.VMEM((C, D), jnp.float32),
            pltpu.VMEM((C, D), jnp.float32),
            pltpu.SemaphoreType.DMA,
            pltpu.SemaphoreType.DMA,
        ],
    )
    return f(x, idx)


def kernel(x, cell_type_indices, permutations):
    idx = permutations.reshape(-1).astype(jnp.int32)
    idx = jnp.concatenate([idx, jnp.zeros((NPAD - N,), jnp.int32)])
    return _gather(x, idx)
